# Initial kernel scaffold; baseline (speedup 1.0000x reference)
import jax
import jax.numpy as jnp
from jax.experimental import pallas as pl


def kernel(nodes, to_neighs, features_table, num_sample):
    raise NotImplementedError("write your pallas kernel here")


# speedup vs baseline: 18.9580x; 18.9580x over previous
import functools

import jax
import jax.numpy as jnp
from jax import lax
from jax.experimental import pallas as pl
from jax.experimental.pallas import tpu as pltpu
from jax.experimental.pallas import tpu_sc as plsc

_L = 16


def _rowmax_body(tbl_ref, out_ref):
    out_ref[...] = jnp.max(tbl_ref[...], axis=1, keepdims=True)


def _bcast_body(s_ref, out_ref):
    out_ref[...] = jnp.broadcast_to(s_ref[...], out_ref.shape)


def _make_sc_gather_max(v_pad, b_pad, num_sample, nw, nc):
    bpw = b_pad // nw
    nchunk = bpw // _L
    mesh = plsc.VectorSubcoreMesh(core_axis_name="c", subcore_axis_name="s")

    @functools.partial(
        pl.kernel,
        mesh=mesh,
        compiler_params=pltpu.CompilerParams(needs_layout_passes=False),
        out_type=jax.ShapeDtypeStruct((b_pad,), jnp.float32),
        scratch_types=[
            pltpu.VMEM((v_pad,), jnp.float32),
            pltpu.VMEM((num_sample, bpw), jnp.int32),
            pltpu.VMEM((bpw,), jnp.float32),
        ],
    )
    def sc_gather_max(rowmax_hbm, neighs_hbm, out_hbm, rowmax_v, idx_v, out_v):
        wid = lax.axis_index("s") * nc + lax.axis_index("c")
        base = wid * bpw
        pltpu.sync_copy(rowmax_hbm, rowmax_v)
        pltpu.sync_copy(neighs_hbm.at[:, pl.ds(base, bpw)], idx_v)

        def chunk_body(c, carry):
            acc = jnp.full((_L,), -jnp.inf, jnp.float32)
            for s in range(num_sample):
                nidx = idx_v[s, pl.ds(c * _L, _L)]
                val = plsc.load_gather(rowmax_v, [nidx])
                acc = jnp.maximum(acc, val)
            out_v[pl.ds(c * _L, _L)] = acc
            return carry

        lax.fori_loop(0, nchunk, chunk_body, 0)
        pltpu.sync_copy(out_v, out_hbm.at[pl.ds(base, bpw)])

    return sc_gather_max


def kernel(nodes, to_neighs, features_table, num_sample):
    del nodes
    n_batch, s = to_neighs.shape
    n_nodes, d = features_table.shape

    info = plsc.get_sparse_core_info()
    nc, ns = info.num_cores, info.num_subcores
    nw = nc * ns
    chunk = nw * 128
    b_pad = ((n_batch + chunk - 1) // chunk) * chunk

    blk = 1000
    row_max = pl.pallas_call(
        _rowmax_body,
        grid=(n_nodes // blk,),
        in_specs=[pl.BlockSpec((blk, d), lambda i: (i, 0))],
        out_specs=pl.BlockSpec((blk, 1), lambda i: (i, 0)),
        out_shape=jax.ShapeDtypeStruct((n_nodes, 1), jnp.float32),
    )(features_table)

    neighs_t = jnp.transpose(to_neighs)
    if b_pad != n_batch:
        neighs_t = jnp.pad(neighs_t, ((0, 0), (0, b_pad - n_batch)))
    v_pad = ((n_nodes + 127) // 128) * 128
    rm_flat = row_max.reshape(n_nodes)
    if v_pad != n_nodes:
        rm_flat = jnp.pad(rm_flat, (0, v_pad - n_nodes))
    sc_fn = _make_sc_gather_max(v_pad, b_pad, s, nw, nc)
    out_scalar = sc_fn(rm_flat, neighs_t)
    out_scalar = out_scalar[:n_batch].reshape(n_batch, 1)

    return pl.pallas_call(
        _bcast_body,
        grid=(n_batch // blk,),
        in_specs=[pl.BlockSpec((blk, 1), lambda i: (i, 0))],
        out_specs=pl.BlockSpec((blk, d), lambda i: (i, 0)),
        out_shape=jax.ShapeDtypeStruct((n_batch, d), jnp.float32),
    )(out_scalar)


# speedup vs baseline: 25.8758x; 1.3649x over previous
import functools

import jax
import jax.numpy as jnp
from jax import lax
from jax.experimental import pallas as pl
from jax.experimental.pallas import tpu as pltpu
from jax.experimental.pallas import tpu_sc as plsc

_L = 16


def _ident128():
    return (
        lax.broadcasted_iota(jnp.int32, (128, 128), 0)
        == lax.broadcasted_iota(jnp.int32, (128, 128), 1)
    ).astype(jnp.float32)


def _rowmax_body(tbl_ref, out_ref):
    col = jnp.max(tbl_ref[...], axis=1, keepdims=True)
    c3 = col.reshape(out_ref.shape[0], 128, 1)
    ident = _ident128()
    for q in range(out_ref.shape[0]):
        row = lax.dot_general(
            c3[q], ident, (((0,), (0,)), ((), ())),
            precision=lax.Precision.HIGHEST,
            preferred_element_type=jnp.float32,
        )
        out_ref[pl.ds(q, 1), :] = row


def _bcast_body(s_ref, out_ref):
    s = s_ref[...]
    ident = _ident128()
    d = out_ref.shape[1]
    for q in range(s.shape[0]):
        colq = lax.dot_general(
            ident, s[q : q + 1, :], (((1,), (1,)), ((), ())),
            precision=lax.Precision.HIGHEST,
            preferred_element_type=jnp.float32,
        )
        out_ref[pl.ds(q * 128, 128), :] = jnp.broadcast_to(colq, (128, d))


def _make_sc_gather_max(v_pad, b_pad, num_sample, nw, nc):
    bpw = b_pad // nw
    nchunk = bpw // _L
    mesh = plsc.VectorSubcoreMesh(core_axis_name="c", subcore_axis_name="s")

    @functools.partial(
        pl.kernel,
        mesh=mesh,
        compiler_params=pltpu.CompilerParams(needs_layout_passes=False),
        out_type=jax.ShapeDtypeStruct((b_pad,), jnp.float32),
        scratch_types=[
            pltpu.VMEM((v_pad // 128, 128), jnp.float32),
            pltpu.VMEM((num_sample, bpw), jnp.int32),
            pltpu.VMEM((bpw,), jnp.float32),
        ],
    )
    def sc_gather_max(rowmax_hbm, neighs_hbm, out_hbm, rowmax_v, idx_v, out_v):
        wid = lax.axis_index("s") * nc + lax.axis_index("c")
        base = wid * bpw
        pltpu.sync_copy(rowmax_hbm, rowmax_v)
        pltpu.sync_copy(neighs_hbm.at[:, pl.ds(base, bpw)], idx_v)

        def chunk_body(c, carry):
            acc = jnp.full((_L,), -jnp.inf, jnp.float32)
            for s in range(num_sample):
                nidx = idx_v[s, pl.ds(c * _L, _L)]
                val = plsc.load_gather(rowmax_v, [nidx >> 7, nidx & 127])
                acc = jnp.maximum(acc, val)
            out_v[pl.ds(c * _L, _L)] = acc
            return carry

        lax.fori_loop(0, nchunk, chunk_body, 0)
        pltpu.sync_copy(out_v, out_hbm.at[pl.ds(base, bpw)])

    return sc_gather_max


def kernel(nodes, to_neighs, features_table, num_sample):
    del nodes
    n_batch, s = to_neighs.shape
    n_nodes, d = features_table.shape

    info = plsc.get_sparse_core_info()
    nw = info.num_cores * info.num_subcores

    blk = 1024
    grid_v = (n_nodes + blk - 1) // blk
    v_pad = grid_v * blk

    row_max = pl.pallas_call(
        _rowmax_body,
        grid=(grid_v,),
        in_specs=[pl.BlockSpec((blk, d), lambda i: (i, 0))],
        out_specs=pl.BlockSpec((blk // 128, 128), lambda i: (i, 0)),
        out_shape=jax.ShapeDtypeStruct((v_pad // 128, 128), jnp.float32),
    )(features_table)

    chunk = nw * 128
    b_pad = ((n_batch + chunk - 1) // chunk) * chunk
    neighs_t = jnp.transpose(to_neighs)
    if b_pad != n_batch:
        neighs_t = jnp.pad(neighs_t, ((0, 0), (0, b_pad - n_batch)))
    sc_fn = _make_sc_gather_max(v_pad, b_pad, s, nw, info.num_cores)
    out_scalar = sc_fn(row_max, neighs_t)

    grid_b = (n_batch + blk - 1) // blk
    scal2d = out_scalar.reshape(b_pad // 128, 128)
    return pl.pallas_call(
        _bcast_body,
        grid=(grid_b,),
        in_specs=[pl.BlockSpec((blk // 128, 128), lambda i: (i, 0))],
        out_specs=pl.BlockSpec((blk, d), lambda i: (i, 0)),
        out_shape=jax.ShapeDtypeStruct((n_batch, d), jnp.float32),
    )(scal2d)


# speedup vs baseline: 29.5357x; 1.1414x over previous
import functools

import jax
import jax.numpy as jnp
from jax import lax
from jax.experimental import pallas as pl
from jax.experimental.pallas import tpu as pltpu
from jax.experimental.pallas import tpu_sc as plsc

_L = 16


def _ident128():
    return (
        lax.broadcasted_iota(jnp.int32, (128, 128), 0)
        == lax.broadcasted_iota(jnp.int32, (128, 128), 1)
    ).astype(jnp.float32)


def _rowmax_body(tbl_ref, out_ref):
    col = jnp.max(tbl_ref[...], axis=1, keepdims=True)
    c3 = col.reshape(out_ref.shape[0], 128, 1)
    ident = _ident128()
    for q in range(out_ref.shape[0]):
        row = lax.dot_general(
            c3[q], ident, (((0,), (0,)), ((), ())),
            precision=lax.Precision.HIGHEST,
            preferred_element_type=jnp.float32,
        )
        out_ref[pl.ds(q, 1), :] = row


def _make_sc_body(v_pad, n_batch, d, num_sample, bpw, batch_rows, nc):
    nbatches = bpw // batch_rows
    chunks_per_batch = batch_rows // _L
    rem = n_batch % batch_rows
    mesh = plsc.VectorSubcoreMesh(core_axis_name="c", subcore_axis_name="s")

    @functools.partial(
        pl.kernel,
        mesh=mesh,
        compiler_params=pltpu.CompilerParams(needs_layout_passes=False),
        out_type=jax.ShapeDtypeStruct((n_batch, d), jnp.float32),
        scratch_types=[
            pltpu.VMEM((v_pad // 128, 128), jnp.float32),
            pltpu.VMEM((num_sample, bpw), jnp.int32),
            pltpu.VMEM((batch_rows, d), jnp.float32),
            pltpu.VMEM((_L,), jnp.float32),
        ],
    )
    def sc_body(rowmax_hbm, neighs_hbm, out_hbm, rm_v, idx_v, out_v, stage_v):
        wid = lax.axis_index("s") * nc + lax.axis_index("c")
        base = wid * bpw
        pltpu.sync_copy(rowmax_hbm, rm_v)
        pltpu.sync_copy(neighs_hbm.at[:, pl.ds(base, bpw)], idx_v)

        def batch_body(g, carry):
            row0 = base + g * batch_rows
            for k in range(chunks_per_batch):
                r0 = k * _L
                acc = jnp.full((_L,), -jnp.inf, jnp.float32)
                for s in range(num_sample):
                    nidx = idx_v[s, pl.ds(g * batch_rows + r0, _L)]
                    val = plsc.load_gather(rm_v, [nidx >> 7, nidx & 127])
                    acc = jnp.maximum(acc, val)
                for r in range(_L):
                    spl = jnp.broadcast_to(acc[r], (_L,))
                    for q in range(d // _L):
                        out_v[r0 + r, pl.ds(q * _L, _L)] = spl

            full = row0 + batch_rows <= n_batch

            @pl.when(full)
            def _():
                pltpu.sync_copy(out_v, out_hbm.at[pl.ds(row0, batch_rows)])

            if rem:
                @pl.when(jnp.logical_and(row0 < n_batch, jnp.logical_not(full)))
                def _():
                    pltpu.sync_copy(
                        out_v.at[pl.ds(0, rem)], out_hbm.at[pl.ds(row0, rem)]
                    )
            return carry

        lax.fori_loop(0, nbatches, batch_body, 0)

    return sc_body


def kernel(nodes, to_neighs, features_table, num_sample):
    del nodes
    n_batch, s = to_neighs.shape
    n_nodes, d = features_table.shape

    info = plsc.get_sparse_core_info()
    nw = info.num_cores * info.num_subcores

    blk = 1024
    grid_v = (n_nodes + blk - 1) // blk
    v_pad = grid_v * blk

    row_max = pl.pallas_call(
        _rowmax_body,
        grid=(grid_v,),
        in_specs=[pl.BlockSpec((blk, d), lambda i: (i, 0))],
        out_specs=pl.BlockSpec((blk // 128, 128), lambda i: (i, 0)),
        out_shape=jax.ShapeDtypeStruct((v_pad // 128, 128), jnp.float32),
    )(features_table)

    batch_rows = 128
    chunk = nw * batch_rows
    b_pad = ((n_batch + chunk - 1) // chunk) * chunk
    bpw = b_pad // nw
    neighs_t = jnp.transpose(to_neighs)
    if b_pad != n_batch:
        neighs_t = jnp.pad(neighs_t, ((0, 0), (0, b_pad - n_batch)))
    sc_fn = _make_sc_body(v_pad, n_batch, d, s, bpw, batch_rows, info.num_cores)
    return sc_fn(row_max, neighs_t)
